# baseline (speedup 1.0000x reference)
import functools

import jax
import jax.numpy as jnp
from jax import lax
from jax.experimental import pallas as pl
from jax.experimental.pallas import tpu as pltpu
from jax.experimental.pallas import tpu_sc as plsc

N = 10000
E = 320000
D = 128
G = 64

NC, NS = 2, 16
NW = NC * NS

CH = 128
NCH = 80
EPT = NCH * CH
EPAD = NW * EPT
TOTCH = EPAD // CH
CA = TOTCH // NW
SRC_ROWS = TOTCH + 128

NP = 10112
RPT = NP // NS

TOTPC = 128
CPA = 4
NBP = TOTPC * CH
RID_ROWS = TOTPC + 32
GP = 128

BM = 400
NB = N // BM

_mesh = plsc.VectorSubcoreMesh(
    core_axis_name="c", subcore_axis_name="s", num_cores=NC, num_subcores=NS)


def _gather_scatter_loop(u_hbm, dst_hbm, src_all, acc, cb, nch,
                         idxd, rows, dsems, gsems, ssems):
    @pl.loop(0, nch)
    def _(c):
        pltpu.sync_copy(dst_hbm.at[pl.ds((cb + c) * CH, CH)], idxd[0])
        pltpu.async_copy(u_hbm.at[src_all.at[c]], rows[0], gsems[0]).wait()
        pltpu.sync_copy(rows[0], acc.at[idxd[0]], add=True)


def _agg_body(u_hbm, src_hbm, dst_hbm, zeros_hbm, part_hbm,
              src_all, idx_d0, idx_d1, rows0, rows1, acc,
              d0, d1, g0, g1, s0, s1):
    cid = lax.axis_index("c")
    sid = lax.axis_index("s")
    w = cid * NS + sid
    cb = w * CA
    pltpu.sync_copy(zeros_hbm.at[pl.ds(sid * RPT, RPT)],
                    acc.at[pl.ds(sid * RPT, RPT)])
    pltpu.sync_copy(src_hbm.at[pl.ds(cb, CA)], src_all)
    plsc.subcore_barrier()

    _gather_scatter_loop(u_hbm, dst_hbm, src_all, acc, cb, CA,
                         (idx_d0, idx_d1), (rows0, rows1), (d0, d1),
                         (g0, g1), (s0, s1))

    plsc.subcore_barrier()
    pltpu.sync_copy(acc.at[pl.ds(sid * RPT, RPT)],
                    part_hbm.at[pl.ds(cid * NP + sid * RPT, RPT)])


_sc_aggregate = functools.partial(
    pl.kernel, _agg_body,
    out_type=jax.ShapeDtypeStruct((NC * NP, D), jnp.float32),
    mesh=_mesh,
    scratch_types=[
        pltpu.VMEM((CA, CH), jnp.int32),
        pltpu.VMEM((CH,), jnp.int32),
        pltpu.VMEM((CH,), jnp.int32),
        pltpu.VMEM((CH, D), jnp.float32),
        pltpu.VMEM((CH, D), jnp.float32),
        pltpu.VMEM_SHARED((NP, D), jnp.float32),
        pltpu.SemaphoreType.DMA,
        pltpu.SemaphoreType.DMA,
        pltpu.SemaphoreType.DMA,
        pltpu.SemaphoreType.DMA,
        pltpu.SemaphoreType.DMA,
        pltpu.SemaphoreType.DMA,
    ],
)()


def _deg_body(dst_hbm, batch_hbm, ones_hbm, zeros_hbm,
              degp_hbm, cntp_hbm, idx0, idx1, ones_v, dacc, cacc, d0, d1):
    cid = lax.axis_index("c")
    sid = lax.axis_index("s")
    w = cid * NS + sid
    pltpu.sync_copy(ones_hbm, ones_v)
    pltpu.sync_copy(zeros_hbm.at[pl.ds(sid * RPT, RPT)],
                    dacc.at[pl.ds(sid * RPT, RPT)])
    pltpu.sync_copy(zeros_hbm.at[pl.ds(sid * 8, 8)],
                    cacc.at[pl.ds(sid * 8, 8)])
    plsc.subcore_barrier()

    idxd = (idx0, idx1)
    sems = (d0, d1)
    pltpu.async_copy(dst_hbm.at[pl.ds(w * EPT, CH)], idx0, d0)

    @pl.loop(0, NCH, step=2)
    def _(c0):
        for b in range(2):
            c = c0 + b
            nxt = c + 1
            o = 1 - b

            @pl.when(nxt < NCH)
            def _():
                pltpu.async_copy(dst_hbm.at[pl.ds((w * NCH + nxt) * CH, CH)],
                                 idxd[o], sems[o])

            pltpu.make_async_copy(dst_hbm.at[pl.ds((w * NCH + c) * CH, CH)],
                                  idxd[b], sems[b]).wait()
            pltpu.sync_copy(ones_v, dacc.at[idxd[b]], add=True)

    def node_chunk(c, carry):
        off = (w * (TOTPC // NW) + c) * CH
        pltpu.sync_copy(batch_hbm.at[pl.ds(off, CH)], idx0)
        pltpu.sync_copy(ones_v, cacc.at[idx0], add=True)
        return carry
    lax.fori_loop(0, TOTPC // NW, node_chunk, 0)

    plsc.subcore_barrier()
    pltpu.sync_copy(dacc.at[pl.ds(sid * RPT, RPT)],
                    degp_hbm.at[pl.ds(cid * NP + sid * RPT, RPT)])
    pltpu.sync_copy(cacc.at[pl.ds(sid * 8, 8)],
                    cntp_hbm.at[pl.ds(cid * GP + sid * 8, 8)])


_sc_degree = functools.partial(
    pl.kernel, _deg_body,
    out_type=(jax.ShapeDtypeStruct((NC * NP, D), jnp.float32),
              jax.ShapeDtypeStruct((NC * GP, D), jnp.float32)),
    mesh=_mesh,
    scratch_types=[
        pltpu.VMEM((CH,), jnp.int32),
        pltpu.VMEM((CH,), jnp.int32),
        pltpu.VMEM((CH, D), jnp.float32),
        pltpu.VMEM_SHARED((NP, D), jnp.float32),
        pltpu.VMEM_SHARED((GP, D), jnp.float32),
        pltpu.SemaphoreType.DMA,
        pltpu.SemaphoreType.DMA,
    ],
)()


def _pool_body(x_hbm, rid_hbm, batch_hbm, zeros_hbm, part_hbm,
               src_all, idx_d0, idx_d1, rows0, rows1, acc,
               d0, d1, g0, g1, s0, s1):
    cid = lax.axis_index("c")
    sid = lax.axis_index("s")
    w = cid * NS + sid
    cb = w * CPA
    pltpu.sync_copy(zeros_hbm.at[pl.ds(sid * 8, 8)],
                    acc.at[pl.ds(sid * 8, 8)])
    pltpu.sync_copy(rid_hbm.at[pl.ds(cb, CPA)], src_all)
    plsc.subcore_barrier()

    _gather_scatter_loop(x_hbm, batch_hbm, src_all, acc, cb, CPA,
                         (idx_d0, idx_d1), (rows0, rows1), (d0, d1),
                         (g0, g1), (s0, s1))

    plsc.subcore_barrier()
    pltpu.sync_copy(acc.at[pl.ds(sid * 8, 8)],
                    part_hbm.at[pl.ds(cid * GP + sid * 8, 8)])


_sc_pool = functools.partial(
    pl.kernel, _pool_body,
    out_type=jax.ShapeDtypeStruct((NC * GP, D), jnp.float32),
    mesh=_mesh,
    scratch_types=[
        pltpu.VMEM((CPA, CH), jnp.int32),
        pltpu.VMEM((CH,), jnp.int32),
        pltpu.VMEM((CH,), jnp.int32),
        pltpu.VMEM((CH, D), jnp.float32),
        pltpu.VMEM((CH, D), jnp.float32),
        pltpu.VMEM_SHARED((GP, D), jnp.float32),
        pltpu.SemaphoreType.DMA,
        pltpu.SemaphoreType.DMA,
        pltpu.SemaphoreType.DMA,
        pltpu.SemaphoreType.DMA,
        pltpu.SemaphoreType.DMA,
        pltpu.SemaphoreType.DMA,
    ],
)()


def _b_body(x_ref, w_ref, d0_ref, d1_ref, dinv_ref, u_ref):
    deg = d0_ref[:, 0:1] + d1_ref[:, 0:1] + 1.0
    dinv = lax.rsqrt(jnp.maximum(deg, 1e-12))
    dinv_ref[...] = dinv
    h = jnp.dot(x_ref[...], w_ref[...], preferred_element_type=jnp.float32)
    u_ref[...] = h * dinv


_tc_prep = pl.pallas_call(
    _b_body,
    grid=(NB,),
    in_specs=[
        pl.BlockSpec((BM, D), lambda i: (i, 0)),
        pl.BlockSpec((D, D), lambda i: (0, 0)),
        pl.BlockSpec((None, BM, D), lambda i: (0, i, 0)),
        pl.BlockSpec((None, BM, D), lambda i: (1, i, 0)),
    ],
    out_specs=[
        pl.BlockSpec((BM, 1), lambda i: (i, 0)),
        pl.BlockSpec((BM, D), lambda i: (i, 0)),
    ],
    out_shape=[
        jax.ShapeDtypeStruct((N, 1), jnp.float32),
        jax.ShapeDtypeStruct((N, D), jnp.float32),
    ],
)


def _d1_body(p0_ref, p1_ref, u_ref, dinv_ref, b_ref, y_ref, st_ref):
    y = jnp.maximum(
        dinv_ref[...] * (p0_ref[...] + p1_ref[...] + u_ref[...]) + b_ref[...],
        0.0)
    y_ref[...] = y

    @pl.when(pl.program_id(0) == 0)
    def _():
        st_ref[...] = jnp.zeros_like(st_ref)
    st_ref[0:1, :] += jnp.sum(y, axis=0, keepdims=True)
    st_ref[1:2, :] += jnp.sum(y * y, axis=0, keepdims=True)


_tc_conv_out = pl.pallas_call(
    _d1_body,
    grid=(NB,),
    in_specs=[
        pl.BlockSpec((None, BM, D), lambda i: (0, i, 0)),
        pl.BlockSpec((None, BM, D), lambda i: (1, i, 0)),
        pl.BlockSpec((BM, D), lambda i: (i, 0)),
        pl.BlockSpec((BM, 1), lambda i: (i, 0)),
        pl.BlockSpec((1, D), lambda i: (0, 0)),
    ],
    out_specs=[
        pl.BlockSpec((BM, D), lambda i: (i, 0)),
        pl.BlockSpec((8, D), lambda i: (0, 0)),
    ],
    out_shape=[
        jax.ShapeDtypeStruct((N, D), jnp.float32),
        jax.ShapeDtypeStruct((8, D), jnp.float32),
    ],
)


def _make_bn_apply(residual, matmul):
    def body(*refs):
        it = iter(refs)
        y_ref = next(it)
        st_ref = next(it)
        g_ref = next(it)
        be_ref = next(it)
        r_ref = next(it) if residual else None
        if matmul:
            dinv_ref = next(it)
            w_ref = next(it)
        x_ref = next(it)
        u_ref = next(it) if matmul else None

        mean = st_ref[0:1, :] * (1.0 / N)
        var = st_ref[1:2, :] * (1.0 / N) - mean * mean
        rstd = lax.rsqrt(var + 1e-5)
        xl = (y_ref[...] - mean) * rstd * g_ref[...] + be_ref[...]
        if residual:
            xl = xl + r_ref[...]
        x_ref[...] = xl
        if matmul:
            h = jnp.dot(xl, w_ref[...], preferred_element_type=jnp.float32)
            u_ref[...] = h * dinv_ref[...]

    in_specs = [
        pl.BlockSpec((BM, D), lambda i: (i, 0)),
        pl.BlockSpec((8, D), lambda i: (0, 0)),
        pl.BlockSpec((1, D), lambda i: (0, 0)),
        pl.BlockSpec((1, D), lambda i: (0, 0)),
    ]
    if residual:
        in_specs.append(pl.BlockSpec((BM, D), lambda i: (i, 0)))
    if matmul:
        in_specs.append(pl.BlockSpec((BM, 1), lambda i: (i, 0)))
        in_specs.append(pl.BlockSpec((D, D), lambda i: (0, 0)))
    out_specs = [pl.BlockSpec((BM, D), lambda i: (i, 0))]
    out_shape = [jax.ShapeDtypeStruct((N, D), jnp.float32)]
    if matmul:
        out_specs.append(pl.BlockSpec((BM, D), lambda i: (i, 0)))
        out_shape.append(jax.ShapeDtypeStruct((N, D), jnp.float32))
    return pl.pallas_call(body, grid=(NB,), in_specs=in_specs,
                          out_specs=out_specs, out_shape=out_shape)


_tc_bn_mm = _make_bn_apply(residual=False, matmul=True)
_tc_bn_res_mm = _make_bn_apply(residual=True, matmul=True)
_tc_bn_res = _make_bn_apply(residual=True, matmul=False)


def _head_body(s0_ref, s1_ref, c0_ref, c1_ref, lw1_ref, lb1_ref,
               lw2_ref, lb2_ref, out_ref):
    cnt = c0_ref[:, 0:1] + c1_ref[:, 0:1]
    pooled = (s0_ref[...] + s1_ref[...]) / jnp.maximum(cnt, 1.0)
    h = jnp.maximum(
        jnp.dot(pooled, lw1_ref[...], preferred_element_type=jnp.float32)
        + lb1_ref[...], 0.0)
    out_ref[...] = (
        jnp.dot(h, lw2_ref[...], preferred_element_type=jnp.float32)
        + lb2_ref[...])


_tc_head = pl.pallas_call(
    _head_body,
    grid=(1,),
    in_specs=[
        pl.BlockSpec((None, G, D), lambda i: (0, 0, 0)),
        pl.BlockSpec((None, G, D), lambda i: (1, 0, 0)),
        pl.BlockSpec((None, G, D), lambda i: (0, 0, 0)),
        pl.BlockSpec((None, G, D), lambda i: (1, 0, 0)),
        pl.BlockSpec((D, D), lambda i: (0, 0)),
        pl.BlockSpec((1, D), lambda i: (0, 0)),
        pl.BlockSpec((D, 1), lambda i: (0, 0)),
        pl.BlockSpec((1, 1), lambda i: (0, 0)),
    ],
    out_specs=pl.BlockSpec((G, 1), lambda i: (0, 0)),
    out_shape=jax.ShapeDtypeStruct((G, 1), jnp.float32),
)


def kernel(x, edge_index, batch, W1, b1, W2, b2, W3, b3,
           g1, be1, g2, be2, g3, be3, lw1, lb1, lw2, lb2):
    src = edge_index[0]
    dst = edge_index[1]
    pad_e = EPAD - E
    src_pad = jnp.concatenate(
        [src, jnp.zeros((pad_e + 128 * CH,), jnp.int32)])
    src_pad = src_pad.reshape(SRC_ROWS, CH)
    dst_pad = jnp.concatenate([dst, jnp.full((pad_e,), N, jnp.int32)])
    batch_pad = jnp.concatenate([batch, jnp.full((NBP - N,), G, jnp.int32)])
    rowids = jnp.concatenate([jnp.arange(N, dtype=jnp.int32),
                              jnp.zeros((RID_ROWS * CH - N,), jnp.int32)])
    rowids = rowids.reshape(RID_ROWS, CH)
    zeros128 = jnp.zeros((NP, D), jnp.float32)
    ones128 = jnp.ones((CH, D), jnp.float32)

    b1r, b2r, b3r = (v.reshape(1, D) for v in (b1, b2, b3))
    g1r, g2r, g3r = (v.reshape(1, D) for v in (g1, g2, g3))
    be1r, be2r, be3r = (v.reshape(1, D) for v in (be1, be2, be3))
    lb1r = lb1.reshape(1, D)
    lb2r = lb2.reshape(1, 1)

    degp, cntp = _sc_degree(dst_pad, batch_pad, ones128, zeros128)
    degp = degp.reshape(NC, NP, D)
    cntp = cntp.reshape(NC, GP, D)
    dinv, u1 = _tc_prep(x, W1, degp, degp)

    p1 = _sc_aggregate(u1, src_pad, dst_pad, zeros128).reshape(NC, NP, D)
    y1, st1 = _tc_conv_out(p1, p1, u1, dinv, b1r)
    x1, u2 = _tc_bn_mm(y1, st1, g1r, be1r, dinv, W2)

    p2 = _sc_aggregate(u2, src_pad, dst_pad, zeros128).reshape(NC, NP, D)
    y2, st2 = _tc_conv_out(p2, p2, u2, dinv, b2r)
    x2, u3 = _tc_bn_res_mm(y2, st2, g2r, be2r, x1, dinv, W3)

    p3 = _sc_aggregate(u3, src_pad, dst_pad, zeros128).reshape(NC, NP, D)
    y3, st3 = _tc_conv_out(p3, p3, u3, dinv, b3r)
    (x3,) = _tc_bn_res(y3, st3, g3r, be3r, x2)

    poolp = _sc_pool(x3, rowids, batch_pad, zeros128).reshape(NC, GP, D)
    return _tc_head(poolp, poolp, cntp, cntp, lw1, lb1r, lw2, lb2r)


# speedup vs baseline: 1.1225x; 1.1225x over previous
import functools

import jax
import jax.numpy as jnp
from jax import lax
from jax.experimental import pallas as pl
from jax.experimental.pallas import tpu as pltpu
from jax.experimental.pallas import tpu_sc as plsc

N = 10000
E = 320000
D = 128
G = 64

NC, NS = 2, 16
NW = NC * NS

CH = 128
NCH = 80
EPT = NCH * CH
EPAD = NW * EPT
TOTCH = EPAD // CH
CA = TOTCH // NW
SRC_ROWS = TOTCH + 128

NP = 10112
RPT = NP // NS

TOTPC = 128
CPA = 4
NBP = TOTPC * CH
RID_ROWS = TOTPC + 32
GP = 128

BM = 400
NB = N // BM

_mesh = plsc.VectorSubcoreMesh(
    core_axis_name="c", subcore_axis_name="s", num_cores=NC, num_subcores=NS)


def _gather_scatter_loop(u_hbm, dst_hbm, src_hbm, acc, cb, nch,
                         idx_s, idx_d, rows, gsem):
    @pl.loop(0, nch)
    def _(c):
        pltpu.sync_copy(src_hbm.at[pl.ds((cb + c) * CH, CH)], idx_s)
        pltpu.sync_copy(dst_hbm.at[pl.ds((cb + c) * CH, CH)], idx_d)
        pltpu.async_copy(u_hbm.at[idx_s], rows, gsem).wait()
        pltpu.sync_copy(rows, acc.at[idx_d], add=True)


def _agg_body(u_hbm, src_hbm, dst_hbm, zeros_hbm, part_hbm,
              idx_s, idx_d, rows, acc, gsem):
    cid = lax.axis_index("c")
    sid = lax.axis_index("s")
    w = cid * NS + sid
    cb = w * CA
    pltpu.sync_copy(zeros_hbm.at[pl.ds(sid * RPT, RPT)],
                    acc.at[pl.ds(sid * RPT, RPT)])
    plsc.subcore_barrier()

    _gather_scatter_loop(u_hbm, dst_hbm, src_hbm, acc, cb, CA,
                         idx_s, idx_d, rows, gsem)

    plsc.subcore_barrier()
    pltpu.sync_copy(acc.at[pl.ds(sid * RPT, RPT)],
                    part_hbm.at[pl.ds(cid * NP + sid * RPT, RPT)])


_sc_aggregate = functools.partial(
    pl.kernel, _agg_body,
    out_type=jax.ShapeDtypeStruct((NC * NP, D), jnp.float32),
    mesh=_mesh,
    scratch_types=[
        pltpu.VMEM((CH,), jnp.int32),
        pltpu.VMEM((CH,), jnp.int32),
        pltpu.VMEM((CH, D), jnp.float32),
        pltpu.VMEM_SHARED((NP, D), jnp.float32),
        pltpu.SemaphoreType.DMA,
    ],
)()


def _deg_body(dst_hbm, batch_hbm, ones_hbm, zeros_hbm,
              degp_hbm, cntp_hbm, idx0, idx1, ones_v, dacc, cacc, d0, d1):
    cid = lax.axis_index("c")
    sid = lax.axis_index("s")
    w = cid * NS + sid
    pltpu.sync_copy(ones_hbm, ones_v)
    pltpu.sync_copy(zeros_hbm.at[pl.ds(sid * RPT, RPT)],
                    dacc.at[pl.ds(sid * RPT, RPT)])
    pltpu.sync_copy(zeros_hbm.at[pl.ds(sid * 8, 8)],
                    cacc.at[pl.ds(sid * 8, 8)])
    plsc.subcore_barrier()

    idxd = (idx0, idx1)
    sems = (d0, d1)
    pltpu.async_copy(dst_hbm.at[pl.ds(w * EPT, CH)], idx0, d0)

    @pl.loop(0, NCH, step=2)
    def _(c0):
        for b in range(2):
            c = c0 + b
            nxt = c + 1
            o = 1 - b

            @pl.when(nxt < NCH)
            def _():
                pltpu.async_copy(dst_hbm.at[pl.ds((w * NCH + nxt) * CH, CH)],
                                 idxd[o], sems[o])

            pltpu.make_async_copy(dst_hbm.at[pl.ds((w * NCH + c) * CH, CH)],
                                  idxd[b], sems[b]).wait()
            pltpu.sync_copy(ones_v, dacc.at[idxd[b]], add=True)

    def node_chunk(c, carry):
        off = (w * (TOTPC // NW) + c) * CH
        pltpu.sync_copy(batch_hbm.at[pl.ds(off, CH)], idx0)
        pltpu.sync_copy(ones_v, cacc.at[idx0], add=True)
        return carry
    lax.fori_loop(0, TOTPC // NW, node_chunk, 0)

    plsc.subcore_barrier()
    pltpu.sync_copy(dacc.at[pl.ds(sid * RPT, RPT)],
                    degp_hbm.at[pl.ds(cid * NP + sid * RPT, RPT)])
    pltpu.sync_copy(cacc.at[pl.ds(sid * 8, 8)],
                    cntp_hbm.at[pl.ds(cid * GP + sid * 8, 8)])


_sc_degree = functools.partial(
    pl.kernel, _deg_body,
    out_type=(jax.ShapeDtypeStruct((NC * NP, D), jnp.float32),
              jax.ShapeDtypeStruct((NC * GP, D), jnp.float32)),
    mesh=_mesh,
    scratch_types=[
        pltpu.VMEM((CH,), jnp.int32),
        pltpu.VMEM((CH,), jnp.int32),
        pltpu.VMEM((CH, D), jnp.float32),
        pltpu.VMEM_SHARED((NP, D), jnp.float32),
        pltpu.VMEM_SHARED((GP, D), jnp.float32),
        pltpu.SemaphoreType.DMA,
        pltpu.SemaphoreType.DMA,
    ],
)()


def _pool_body(x_hbm, rid_hbm, batch_hbm, zeros_hbm, part_hbm,
               idx_s, idx_d, rows, acc, gsem):
    cid = lax.axis_index("c")
    sid = lax.axis_index("s")
    w = cid * NS + sid
    cb = w * CPA
    pltpu.sync_copy(zeros_hbm.at[pl.ds(sid * 8, 8)],
                    acc.at[pl.ds(sid * 8, 8)])
    plsc.subcore_barrier()

    _gather_scatter_loop(x_hbm, batch_hbm, rid_hbm, acc, cb, CPA,
                         idx_s, idx_d, rows, gsem)

    plsc.subcore_barrier()
    pltpu.sync_copy(acc.at[pl.ds(sid * 8, 8)],
                    part_hbm.at[pl.ds(cid * GP + sid * 8, 8)])


_sc_pool = functools.partial(
    pl.kernel, _pool_body,
    out_type=jax.ShapeDtypeStruct((NC * GP, D), jnp.float32),
    mesh=_mesh,
    scratch_types=[
        pltpu.VMEM((CH,), jnp.int32),
        pltpu.VMEM((CH,), jnp.int32),
        pltpu.VMEM((CH, D), jnp.float32),
        pltpu.VMEM_SHARED((GP, D), jnp.float32),
        pltpu.SemaphoreType.DMA,
    ],
)()


def _b_body(x_ref, w_ref, d0_ref, d1_ref, dinv_ref, u_ref):
    deg = d0_ref[:, 0:1] + d1_ref[:, 0:1] + 1.0
    dinv = lax.rsqrt(jnp.maximum(deg, 1e-12))
    dinv_ref[...] = dinv
    h = jnp.dot(x_ref[...], w_ref[...], preferred_element_type=jnp.float32)
    u_ref[...] = h * dinv


_tc_prep = pl.pallas_call(
    _b_body,
    grid=(NB,),
    in_specs=[
        pl.BlockSpec((BM, D), lambda i: (i, 0)),
        pl.BlockSpec((D, D), lambda i: (0, 0)),
        pl.BlockSpec((None, BM, D), lambda i: (0, i, 0)),
        pl.BlockSpec((None, BM, D), lambda i: (1, i, 0)),
    ],
    out_specs=[
        pl.BlockSpec((BM, 1), lambda i: (i, 0)),
        pl.BlockSpec((BM, D), lambda i: (i, 0)),
    ],
    out_shape=[
        jax.ShapeDtypeStruct((N, 1), jnp.float32),
        jax.ShapeDtypeStruct((N, D), jnp.float32),
    ],
)


def _d1_body(p0_ref, p1_ref, u_ref, dinv_ref, b_ref, y_ref, st_ref):
    y = jnp.maximum(
        dinv_ref[...] * (p0_ref[...] + p1_ref[...] + u_ref[...]) + b_ref[...],
        0.0)
    y_ref[...] = y

    @pl.when(pl.program_id(0) == 0)
    def _():
        st_ref[...] = jnp.zeros_like(st_ref)
    st_ref[0:1, :] += jnp.sum(y, axis=0, keepdims=True)
    st_ref[1:2, :] += jnp.sum(y * y, axis=0, keepdims=True)


_tc_conv_out = pl.pallas_call(
    _d1_body,
    grid=(NB,),
    in_specs=[
        pl.BlockSpec((None, BM, D), lambda i: (0, i, 0)),
        pl.BlockSpec((None, BM, D), lambda i: (1, i, 0)),
        pl.BlockSpec((BM, D), lambda i: (i, 0)),
        pl.BlockSpec((BM, 1), lambda i: (i, 0)),
        pl.BlockSpec((1, D), lambda i: (0, 0)),
    ],
    out_specs=[
        pl.BlockSpec((BM, D), lambda i: (i, 0)),
        pl.BlockSpec((8, D), lambda i: (0, 0)),
    ],
    out_shape=[
        jax.ShapeDtypeStruct((N, D), jnp.float32),
        jax.ShapeDtypeStruct((8, D), jnp.float32),
    ],
)


def _make_bn_apply(residual, matmul):
    def body(*refs):
        it = iter(refs)
        y_ref = next(it)
        st_ref = next(it)
        g_ref = next(it)
        be_ref = next(it)
        r_ref = next(it) if residual else None
        if matmul:
            dinv_ref = next(it)
            w_ref = next(it)
        x_ref = next(it)
        u_ref = next(it) if matmul else None

        mean = st_ref[0:1, :] * (1.0 / N)
        var = st_ref[1:2, :] * (1.0 / N) - mean * mean
        rstd = lax.rsqrt(var + 1e-5)
        xl = (y_ref[...] - mean) * rstd * g_ref[...] + be_ref[...]
        if residual:
            xl = xl + r_ref[...]
        x_ref[...] = xl
        if matmul:
            h = jnp.dot(xl, w_ref[...], preferred_element_type=jnp.float32)
            u_ref[...] = h * dinv_ref[...]

    in_specs = [
        pl.BlockSpec((BM, D), lambda i: (i, 0)),
        pl.BlockSpec((8, D), lambda i: (0, 0)),
        pl.BlockSpec((1, D), lambda i: (0, 0)),
        pl.BlockSpec((1, D), lambda i: (0, 0)),
    ]
    if residual:
        in_specs.append(pl.BlockSpec((BM, D), lambda i: (i, 0)))
    if matmul:
        in_specs.append(pl.BlockSpec((BM, 1), lambda i: (i, 0)))
        in_specs.append(pl.BlockSpec((D, D), lambda i: (0, 0)))
    out_specs = [pl.BlockSpec((BM, D), lambda i: (i, 0))]
    out_shape = [jax.ShapeDtypeStruct((N, D), jnp.float32)]
    if matmul:
        out_specs.append(pl.BlockSpec((BM, D), lambda i: (i, 0)))
        out_shape.append(jax.ShapeDtypeStruct((N, D), jnp.float32))
    return pl.pallas_call(body, grid=(NB,), in_specs=in_specs,
                          out_specs=out_specs, out_shape=out_shape)


_tc_bn_mm = _make_bn_apply(residual=False, matmul=True)
_tc_bn_res_mm = _make_bn_apply(residual=True, matmul=True)
_tc_bn_res = _make_bn_apply(residual=True, matmul=False)


def _head_body(s0_ref, s1_ref, c0_ref, c1_ref, lw1_ref, lb1_ref,
               lw2_ref, lb2_ref, out_ref):
    cnt = c0_ref[:, 0:1] + c1_ref[:, 0:1]
    pooled = (s0_ref[...] + s1_ref[...]) / jnp.maximum(cnt, 1.0)
    h = jnp.maximum(
        jnp.dot(pooled, lw1_ref[...], preferred_element_type=jnp.float32)
        + lb1_ref[...], 0.0)
    out_ref[...] = (
        jnp.dot(h, lw2_ref[...], preferred_element_type=jnp.float32)
        + lb2_ref[...])


_tc_head = pl.pallas_call(
    _head_body,
    grid=(1,),
    in_specs=[
        pl.BlockSpec((None, G, D), lambda i: (0, 0, 0)),
        pl.BlockSpec((None, G, D), lambda i: (1, 0, 0)),
        pl.BlockSpec((None, G, D), lambda i: (0, 0, 0)),
        pl.BlockSpec((None, G, D), lambda i: (1, 0, 0)),
        pl.BlockSpec((D, D), lambda i: (0, 0)),
        pl.BlockSpec((1, D), lambda i: (0, 0)),
        pl.BlockSpec((D, 1), lambda i: (0, 0)),
        pl.BlockSpec((1, 1), lambda i: (0, 0)),
    ],
    out_specs=pl.BlockSpec((G, 1), lambda i: (0, 0)),
    out_shape=jax.ShapeDtypeStruct((G, 1), jnp.float32),
)


def kernel(x, edge_index, batch, W1, b1, W2, b2, W3, b3,
           g1, be1, g2, be2, g3, be3, lw1, lb1, lw2, lb2):
    src = edge_index[0]
    dst = edge_index[1]
    pad_e = EPAD - E
    src_pad = jnp.concatenate([src, jnp.zeros((pad_e,), jnp.int32)])
    dst_pad = jnp.concatenate([dst, jnp.full((pad_e,), N, jnp.int32)])
    batch_pad = jnp.concatenate([batch, jnp.full((NBP - N,), G, jnp.int32)])
    rowids = jnp.concatenate([jnp.arange(N, dtype=jnp.int32),
                              jnp.zeros((NBP - N,), jnp.int32)])
    zeros128 = jnp.zeros((NP, D), jnp.float32)
    ones128 = jnp.ones((CH, D), jnp.float32)

    b1r, b2r, b3r = (v.reshape(1, D) for v in (b1, b2, b3))
    g1r, g2r, g3r = (v.reshape(1, D) for v in (g1, g2, g3))
    be1r, be2r, be3r = (v.reshape(1, D) for v in (be1, be2, be3))
    lb1r = lb1.reshape(1, D)
    lb2r = lb2.reshape(1, 1)

    degp, cntp = _sc_degree(dst_pad, batch_pad, ones128, zeros128)
    degp = degp.reshape(NC, NP, D)
    cntp = cntp.reshape(NC, GP, D)
    dinv, u1 = _tc_prep(x, W1, degp, degp)

    p1 = _sc_aggregate(u1, src_pad, dst_pad, zeros128).reshape(NC, NP, D)
    y1, st1 = _tc_conv_out(p1, p1, u1, dinv, b1r)
    x1, u2 = _tc_bn_mm(y1, st1, g1r, be1r, dinv, W2)

    p2 = _sc_aggregate(u2, src_pad, dst_pad, zeros128).reshape(NC, NP, D)
    y2, st2 = _tc_conv_out(p2, p2, u2, dinv, b2r)
    x2, u3 = _tc_bn_res_mm(y2, st2, g2r, be2r, x1, dinv, W3)

    p3 = _sc_aggregate(u3, src_pad, dst_pad, zeros128).reshape(NC, NP, D)
    y3, st3 = _tc_conv_out(p3, p3, u3, dinv, b3r)
    (x3,) = _tc_bn_res(y3, st3, g3r, be3r, x2)

    poolp = _sc_pool(x3, rowids, batch_pad, zeros128).reshape(NC, GP, D)
    return _tc_head(poolp, poolp, cntp, cntp, lw1, lb1r, lw2, lb2r)


# speedup vs baseline: 1.5052x; 1.3409x over previous
import functools

import jax
import jax.numpy as jnp
from jax import lax
from jax.experimental import pallas as pl
from jax.experimental.pallas import tpu as pltpu
from jax.experimental.pallas import tpu_sc as plsc

N = 10000
E = 320000
D = 128
G = 64

NC, NS = 2, 16
NW = NC * NS

CH = 128
EPT = 10112
NCH = EPT // CH
EPAD = NW * EPT

NP = 12800
RPT = NP // NS

NBP = 12288
BPT = NBP // NW
GP = 128

BM = 400
NB = N // BM

_mesh = plsc.VectorSubcoreMesh(
    core_axis_name="c", subcore_axis_name="s", num_cores=NC, num_subcores=NS)


def _deg_body(dst_hbm, batch_hbm, ones_hbm, zeros_hbm,
              degp_hbm, cntp_hbm, idx, ones_v, dacc, cacc):
    cid = lax.axis_index("c")
    sid = lax.axis_index("s")
    w = cid * NS + sid
    pltpu.sync_copy(ones_hbm, ones_v)
    pltpu.sync_copy(zeros_hbm.at[pl.ds(sid * RPT, RPT)],
                    dacc.at[pl.ds(sid * RPT, RPT)])
    pltpu.sync_copy(zeros_hbm.at[pl.ds(sid * 8, 8)],
                    cacc.at[pl.ds(sid * 8, 8)])
    plsc.subcore_barrier()

    def edge_chunk(c, carry):
        off = (w * NCH + c) * CH
        pltpu.sync_copy(dst_hbm.at[pl.ds(off, CH)], idx)
        pltpu.sync_copy(ones_v, dacc.at[idx], add=True)
        return carry
    lax.fori_loop(0, NCH, edge_chunk, 0)

    def node_chunk(c, carry):
        off = w * BPT + c * CH
        pltpu.sync_copy(batch_hbm.at[pl.ds(off, CH)], idx)
        pltpu.sync_copy(ones_v, cacc.at[idx], add=True)
        return carry
    lax.fori_loop(0, BPT // CH, node_chunk, 0)

    plsc.subcore_barrier()
    pltpu.sync_copy(dacc.at[pl.ds(sid * RPT, RPT)],
                    degp_hbm.at[pl.ds(cid * NP + sid * RPT, RPT)])
    pltpu.sync_copy(cacc.at[pl.ds(sid * 8, 8)],
                    cntp_hbm.at[pl.ds(cid * GP + sid * 8, 8)])


_sc_degree = functools.partial(
    pl.kernel, _deg_body,
    out_type=(jax.ShapeDtypeStruct((NC * NP, D), jnp.float32),
              jax.ShapeDtypeStruct((NC * GP, D), jnp.float32)),
    mesh=_mesh,
    scratch_types=[
        pltpu.VMEM((CH,), jnp.int32),
        pltpu.VMEM((CH, D), jnp.float32),
        pltpu.VMEM_SHARED((NP, D), jnp.float32),
        pltpu.VMEM_SHARED((GP, D), jnp.float32),
    ],
)()


def _agg_body(u_hbm, src_hbm, dst_hbm, zeros_hbm,
              part_hbm, idx_s, idx_d, rows, acc, sem):
    cid = lax.axis_index("c")
    sid = lax.axis_index("s")
    w = cid * NS + sid
    pltpu.sync_copy(zeros_hbm.at[pl.ds(sid * RPT, RPT)],
                    acc.at[pl.ds(sid * RPT, RPT)])
    plsc.subcore_barrier()

    def chunk(c, carry):
        off = (w * NCH + c) * CH
        pltpu.sync_copy(src_hbm.at[pl.ds(off, CH)], idx_s)
        pltpu.sync_copy(dst_hbm.at[pl.ds(off, CH)], idx_d)
        pltpu.async_copy(u_hbm.at[idx_s], rows, sem).wait()
        pltpu.sync_copy(rows, acc.at[idx_d], add=True)
        return carry
    lax.fori_loop(0, NCH, chunk, 0)

    plsc.subcore_barrier()
    pltpu.sync_copy(acc.at[pl.ds(sid * RPT, RPT)],
                    part_hbm.at[pl.ds(cid * NP + sid * RPT, RPT)])


_sc_aggregate = functools.partial(
    pl.kernel, _agg_body,
    out_type=jax.ShapeDtypeStruct((NC * NP, D), jnp.float32),
    mesh=_mesh,
    scratch_types=[
        pltpu.VMEM((CH,), jnp.int32),
        pltpu.VMEM((CH,), jnp.int32),
        pltpu.VMEM((CH, D), jnp.float32),
        pltpu.VMEM_SHARED((NP, D), jnp.float32),
        pltpu.SemaphoreType.DMA,
    ],
)()


def _pool_body(x_hbm, rid_hbm, batch_hbm, zeros_hbm,
               part_hbm, idx_s, idx_d, rows, acc, sem):
    cid = lax.axis_index("c")
    sid = lax.axis_index("s")
    w = cid * NS + sid
    pltpu.sync_copy(zeros_hbm.at[pl.ds(sid * 8, 8)],
                    acc.at[pl.ds(sid * 8, 8)])
    plsc.subcore_barrier()

    def chunk(c, carry):
        off = w * BPT + c * CH
        pltpu.sync_copy(rid_hbm.at[pl.ds(off, CH)], idx_s)
        pltpu.sync_copy(batch_hbm.at[pl.ds(off, CH)], idx_d)
        pltpu.async_copy(x_hbm.at[idx_s], rows, sem).wait()
        pltpu.sync_copy(rows, acc.at[idx_d], add=True)
        return carry
    lax.fori_loop(0, BPT // CH, chunk, 0)

    plsc.subcore_barrier()
    pltpu.sync_copy(acc.at[pl.ds(sid * 8, 8)],
                    part_hbm.at[pl.ds(cid * GP + sid * 8, 8)])


_sc_pool = functools.partial(
    pl.kernel, _pool_body,
    out_type=jax.ShapeDtypeStruct((NC * GP, D), jnp.float32),
    mesh=_mesh,
    scratch_types=[
        pltpu.VMEM((CH,), jnp.int32),
        pltpu.VMEM((CH,), jnp.int32),
        pltpu.VMEM((CH, D), jnp.float32),
        pltpu.VMEM_SHARED((GP, D), jnp.float32),
        pltpu.SemaphoreType.DMA,
    ],
)()


def _b_body(x_ref, w_ref, d0_ref, d1_ref, dinv_ref, u_ref):
    deg = d0_ref[:, 0:1] + d1_ref[:, 0:1] + 1.0
    dinv = lax.rsqrt(jnp.maximum(deg, 1e-12))
    dinv_ref[...] = dinv
    h = jnp.dot(x_ref[...], w_ref[...], preferred_element_type=jnp.float32)
    u_ref[...] = h * dinv


_tc_prep = pl.pallas_call(
    _b_body,
    grid=(NB,),
    in_specs=[
        pl.BlockSpec((BM, D), lambda i: (i, 0)),
        pl.BlockSpec((D, D), lambda i: (0, 0)),
        pl.BlockSpec((BM, D), lambda i: (i, 0)),
        pl.BlockSpec((BM, D), lambda i: (i + NP // BM, 0)),
    ],
    out_specs=[
        pl.BlockSpec((BM, 1), lambda i: (i, 0)),
        pl.BlockSpec((BM, D), lambda i: (i, 0)),
    ],
    out_shape=[
        jax.ShapeDtypeStruct((N, 1), jnp.float32),
        jax.ShapeDtypeStruct((N, D), jnp.float32),
    ],
)


def _d1_body(p0_ref, p1_ref, u_ref, dinv_ref, b_ref, y_ref, st_ref):
    y = jnp.maximum(
        dinv_ref[...] * (p0_ref[...] + p1_ref[...] + u_ref[...]) + b_ref[...],
        0.0)
    y_ref[...] = y

    @pl.when(pl.program_id(0) == 0)
    def _():
        st_ref[...] = jnp.zeros_like(st_ref)
    st_ref[0:1, :] += jnp.sum(y, axis=0, keepdims=True)
    st_ref[1:2, :] += jnp.sum(y * y, axis=0, keepdims=True)


_tc_conv_out = pl.pallas_call(
    _d1_body,
    grid=(NB,),
    in_specs=[
        pl.BlockSpec((BM, D), lambda i: (i, 0)),
        pl.BlockSpec((BM, D), lambda i: (i + NP // BM, 0)),
        pl.BlockSpec((BM, D), lambda i: (i, 0)),
        pl.BlockSpec((BM, 1), lambda i: (i, 0)),
        pl.BlockSpec((1, D), lambda i: (0, 0)),
    ],
    out_specs=[
        pl.BlockSpec((BM, D), lambda i: (i, 0)),
        pl.BlockSpec((8, D), lambda i: (0, 0)),
    ],
    out_shape=[
        jax.ShapeDtypeStruct((N, D), jnp.float32),
        jax.ShapeDtypeStruct((8, D), jnp.float32),
    ],
)


def _make_bn_apply(residual, matmul):
    def body(*refs):
        it = iter(refs)
        y_ref = next(it)
        st_ref = next(it)
        g_ref = next(it)
        be_ref = next(it)
        r_ref = next(it) if residual else None
        if matmul:
            dinv_ref = next(it)
            w_ref = next(it)
        x_ref = next(it)
        u_ref = next(it) if matmul else None

        mean = st_ref[0:1, :] * (1.0 / N)
        var = st_ref[1:2, :] * (1.0 / N) - mean * mean
        rstd = lax.rsqrt(var + 1e-5)
        xl = (y_ref[...] - mean) * rstd * g_ref[...] + be_ref[...]
        if residual:
            xl = xl + r_ref[...]
        x_ref[...] = xl
        if matmul:
            h = jnp.dot(xl, w_ref[...], preferred_element_type=jnp.float32)
            u_ref[...] = h * dinv_ref[...]

    in_specs = [
        pl.BlockSpec((BM, D), lambda i: (i, 0)),
        pl.BlockSpec((8, D), lambda i: (0, 0)),
        pl.BlockSpec((1, D), lambda i: (0, 0)),
        pl.BlockSpec((1, D), lambda i: (0, 0)),
    ]
    if residual:
        in_specs.append(pl.BlockSpec((BM, D), lambda i: (i, 0)))
    if matmul:
        in_specs.append(pl.BlockSpec((BM, 1), lambda i: (i, 0)))
        in_specs.append(pl.BlockSpec((D, D), lambda i: (0, 0)))
    out_specs = [pl.BlockSpec((BM, D), lambda i: (i, 0))]
    out_shape = [jax.ShapeDtypeStruct((N, D), jnp.float32)]
    if matmul:
        out_specs.append(pl.BlockSpec((BM, D), lambda i: (i, 0)))
        out_shape.append(jax.ShapeDtypeStruct((N, D), jnp.float32))
    return pl.pallas_call(body, grid=(NB,), in_specs=in_specs,
                          out_specs=out_specs, out_shape=out_shape)


_tc_bn_mm = _make_bn_apply(residual=False, matmul=True)
_tc_bn_res_mm = _make_bn_apply(residual=True, matmul=True)
_tc_bn_res = _make_bn_apply(residual=True, matmul=False)


def _head_body(s0_ref, s1_ref, c0_ref, c1_ref, lw1_ref, lb1_ref,
               lw2_ref, lb2_ref, out_ref):
    cnt = c0_ref[:, 0:1] + c1_ref[:, 0:1]
    pooled = (s0_ref[...] + s1_ref[...]) / jnp.maximum(cnt, 1.0)
    h = jnp.maximum(
        jnp.dot(pooled, lw1_ref[...], preferred_element_type=jnp.float32)
        + lb1_ref[...], 0.0)
    out_ref[...] = (
        jnp.dot(h, lw2_ref[...], preferred_element_type=jnp.float32)
        + lb2_ref[...])


_tc_head = pl.pallas_call(
    _head_body,
    grid=(1,),
    in_specs=[
        pl.BlockSpec((G, D), lambda i: (0, 0)),
        pl.BlockSpec((G, D), lambda i: (GP // G, 0)),
        pl.BlockSpec((G, D), lambda i: (0, 0)),
        pl.BlockSpec((G, D), lambda i: (GP // G, 0)),
        pl.BlockSpec((D, D), lambda i: (0, 0)),
        pl.BlockSpec((1, D), lambda i: (0, 0)),
        pl.BlockSpec((D, 1), lambda i: (0, 0)),
        pl.BlockSpec((1, 1), lambda i: (0, 0)),
    ],
    out_specs=pl.BlockSpec((G, 1), lambda i: (0, 0)),
    out_shape=jax.ShapeDtypeStruct((G, 1), jnp.float32),
)


def kernel(x, edge_index, batch, W1, b1, W2, b2, W3, b3,
           g1, be1, g2, be2, g3, be3, lw1, lb1, lw2, lb2):
    src = edge_index[0]
    dst = edge_index[1]
    pad_e = EPAD - E
    src_pad = jnp.concatenate([src, jnp.zeros((pad_e,), jnp.int32)])
    dst_pad = jnp.concatenate([dst, jnp.full((pad_e,), N, jnp.int32)])
    batch_pad = jnp.concatenate([batch, jnp.full((NBP - N,), G, jnp.int32)])
    rowids = jnp.concatenate([jnp.arange(N, dtype=jnp.int32),
                              jnp.zeros((NBP - N,), jnp.int32)])
    zeros128 = jnp.zeros((NP, D), jnp.float32)
    ones128 = jnp.ones((CH, D), jnp.float32)

    b1r, b2r, b3r = (v.reshape(1, D) for v in (b1, b2, b3))
    g1r, g2r, g3r = (v.reshape(1, D) for v in (g1, g2, g3))
    be1r, be2r, be3r = (v.reshape(1, D) for v in (be1, be2, be3))
    lb1r = lb1.reshape(1, D)
    lb2r = lb2.reshape(1, 1)

    degp, cntp = _sc_degree(dst_pad, batch_pad, ones128, zeros128)
    dinv, u1 = _tc_prep(x, W1, degp, degp)

    p1 = _sc_aggregate(u1, src_pad, dst_pad, zeros128)
    y1, st1 = _tc_conv_out(p1, p1, u1, dinv, b1r)
    x1, u2 = _tc_bn_mm(y1, st1, g1r, be1r, dinv, W2)

    p2 = _sc_aggregate(u2, src_pad, dst_pad, zeros128)
    y2, st2 = _tc_conv_out(p2, p2, u2, dinv, b2r)
    x2, u3 = _tc_bn_res_mm(y2, st2, g2r, be2r, x1, dinv, W3)

    p3 = _sc_aggregate(u3, src_pad, dst_pad, zeros128)
    y3, st3 = _tc_conv_out(p3, p3, u3, dinv, b3r)
    (x3,) = _tc_bn_res(y3, st3, g3r, be3r, x2)

    poolp = _sc_pool(x3, rowids, batch_pad, zeros128)
    return _tc_head(poolp, poolp, cntp, cntp, lw1, lb1r, lw2, lb2r)


# speedup vs baseline: 1.5332x; 1.0186x over previous
import functools

import jax
import jax.numpy as jnp
from jax import lax
from jax.experimental import pallas as pl
from jax.experimental.pallas import tpu as pltpu
from jax.experimental.pallas import tpu_sc as plsc

N = 10000
E = 320000
D = 128
G = 64

NC, NS = 2, 16
NW = NC * NS

CH = 128
EPT = 10112
NCH = EPT // CH
EPAD = NW * EPT

NP = 12800
RPT = NP // NS

NBP = 12288
BPT = NBP // NW
GP = 128

BM = 400
NB = N // BM

_mesh = plsc.VectorSubcoreMesh(
    core_axis_name="c", subcore_axis_name="s", num_cores=NC, num_subcores=NS)


def _deg_body(dst_hbm, batch_hbm, ones_hbm, zeros_hbm,
              degp_hbm, cntp_hbm, idx0, idx1, ones_v, dacc, cacc, d0, d1):
    cid = lax.axis_index("c")
    sid = lax.axis_index("s")
    w = cid * NS + sid
    pltpu.sync_copy(ones_hbm, ones_v)
    pltpu.sync_copy(zeros_hbm.at[pl.ds(sid * RPT, RPT)],
                    dacc.at[pl.ds(sid * RPT, RPT)])
    pltpu.sync_copy(zeros_hbm.at[pl.ds(sid * 8, 8)],
                    cacc.at[pl.ds(sid * 8, 8)])
    plsc.subcore_barrier()

    idxd = (idx0, idx1)
    sems = (d0, d1)
    pltpu.async_copy(dst_hbm.at[pl.ds(w * NCH * CH, CH)], idx0, d0)

    @pl.loop(0, NCH, step=2)
    def _(c0):
        for b in range(2):
            c = c0 + b
            nxt = c + 1
            o = 1 - b

            @pl.when(c < NCH)
            def _():
                @pl.when(nxt < NCH)
                def _():
                    pltpu.async_copy(
                        dst_hbm.at[pl.ds((w * NCH + nxt) * CH, CH)],
                        idxd[o], sems[o])

                pltpu.make_async_copy(
                    dst_hbm.at[pl.ds((w * NCH + c) * CH, CH)],
                    idxd[b], sems[b]).wait()
                pltpu.sync_copy(ones_v, dacc.at[idxd[b]], add=True)

    def node_chunk(c, carry):
        off = w * BPT + c * CH
        pltpu.sync_copy(batch_hbm.at[pl.ds(off, CH)], idx0)
        pltpu.sync_copy(ones_v, cacc.at[idx0], add=True)
        return carry
    lax.fori_loop(0, BPT // CH, node_chunk, 0)

    plsc.subcore_barrier()
    pltpu.sync_copy(dacc.at[pl.ds(sid * RPT, RPT)],
                    degp_hbm.at[pl.ds(cid * NP + sid * RPT, RPT)])
    pltpu.sync_copy(cacc.at[pl.ds(sid * 8, 8)],
                    cntp_hbm.at[pl.ds(cid * GP + sid * 8, 8)])


_sc_degree = functools.partial(
    pl.kernel, _deg_body,
    out_type=(jax.ShapeDtypeStruct((NC * NP, D), jnp.float32),
              jax.ShapeDtypeStruct((NC * GP, D), jnp.float32)),
    mesh=_mesh,
    scratch_types=[
        pltpu.VMEM((CH,), jnp.int32),
        pltpu.VMEM((CH,), jnp.int32),
        pltpu.VMEM((CH, D), jnp.float32),
        pltpu.VMEM_SHARED((NP, D), jnp.float32),
        pltpu.VMEM_SHARED((GP, D), jnp.float32),
        pltpu.SemaphoreType.DMA,
        pltpu.SemaphoreType.DMA,
    ],
)()


def _agg_body(u_hbm, src_hbm, dst_hbm, zeros_hbm,
              part_hbm, idx_s, idx_d, rows, acc, sem):
    cid = lax.axis_index("c")
    sid = lax.axis_index("s")
    w = cid * NS + sid
    pltpu.sync_copy(zeros_hbm.at[pl.ds(sid * RPT, RPT)],
                    acc.at[pl.ds(sid * RPT, RPT)])
    plsc.subcore_barrier()

    def chunk(c, carry):
        off = (w * NCH + c) * CH
        pltpu.sync_copy(src_hbm.at[pl.ds(off, CH)], idx_s)
        pltpu.sync_copy(dst_hbm.at[pl.ds(off, CH)], idx_d)
        pltpu.async_copy(u_hbm.at[idx_s], rows, sem).wait()
        pltpu.sync_copy(rows, acc.at[idx_d], add=True)
        return carry
    lax.fori_loop(0, NCH, chunk, 0)

    plsc.subcore_barrier()
    pltpu.sync_copy(acc.at[pl.ds(sid * RPT, RPT)],
                    part_hbm.at[pl.ds(cid * NP + sid * RPT, RPT)])


_sc_aggregate = functools.partial(
    pl.kernel, _agg_body,
    out_type=jax.ShapeDtypeStruct((NC * NP, D), jnp.float32),
    mesh=_mesh,
    scratch_types=[
        pltpu.VMEM((CH,), jnp.int32),
        pltpu.VMEM((CH,), jnp.int32),
        pltpu.VMEM((CH, D), jnp.float32),
        pltpu.VMEM_SHARED((NP, D), jnp.float32),
        pltpu.SemaphoreType.DMA,
    ],
)()


def _pool_body(x_hbm, rid_hbm, batch_hbm, zeros_hbm,
               part_hbm, idx_s, idx_d, rows, acc, sem):
    cid = lax.axis_index("c")
    sid = lax.axis_index("s")
    w = cid * NS + sid
    pltpu.sync_copy(zeros_hbm.at[pl.ds(sid * 8, 8)],
                    acc.at[pl.ds(sid * 8, 8)])
    plsc.subcore_barrier()

    def chunk(c, carry):
        off = w * BPT + c * CH
        pltpu.sync_copy(rid_hbm.at[pl.ds(off, CH)], idx_s)
        pltpu.sync_copy(batch_hbm.at[pl.ds(off, CH)], idx_d)
        pltpu.async_copy(x_hbm.at[idx_s], rows, sem).wait()
        pltpu.sync_copy(rows, acc.at[idx_d], add=True)
        return carry
    lax.fori_loop(0, BPT // CH, chunk, 0)

    plsc.subcore_barrier()
    pltpu.sync_copy(acc.at[pl.ds(sid * 8, 8)],
                    part_hbm.at[pl.ds(cid * GP + sid * 8, 8)])


_sc_pool = functools.partial(
    pl.kernel, _pool_body,
    out_type=jax.ShapeDtypeStruct((NC * GP, D), jnp.float32),
    mesh=_mesh,
    scratch_types=[
        pltpu.VMEM((CH,), jnp.int32),
        pltpu.VMEM((CH,), jnp.int32),
        pltpu.VMEM((CH, D), jnp.float32),
        pltpu.VMEM_SHARED((GP, D), jnp.float32),
        pltpu.SemaphoreType.DMA,
    ],
)()


def _b_body(x_ref, w_ref, d0_ref, d1_ref, dinv_ref, u_ref):
    deg = d0_ref[:, 0:1] + d1_ref[:, 0:1] + 1.0
    dinv = lax.rsqrt(jnp.maximum(deg, 1e-12))
    dinv_ref[...] = dinv
    h = jnp.dot(x_ref[...], w_ref[...], preferred_element_type=jnp.float32)
    u_ref[...] = h * dinv


_tc_prep = pl.pallas_call(
    _b_body,
    grid=(NB,),
    in_specs=[
        pl.BlockSpec((BM, D), lambda i: (i, 0)),
        pl.BlockSpec((D, D), lambda i: (0, 0)),
        pl.BlockSpec((BM, D), lambda i: (i, 0)),
        pl.BlockSpec((BM, D), lambda i: (i + NP // BM, 0)),
    ],
    out_specs=[
        pl.BlockSpec((BM, 1), lambda i: (i, 0)),
        pl.BlockSpec((BM, D), lambda i: (i, 0)),
    ],
    out_shape=[
        jax.ShapeDtypeStruct((N, 1), jnp.float32),
        jax.ShapeDtypeStruct((N, D), jnp.float32),
    ],
)


def _d1_body(p0_ref, p1_ref, u_ref, dinv_ref, b_ref, y_ref, st_ref):
    y = jnp.maximum(
        dinv_ref[...] * (p0_ref[...] + p1_ref[...] + u_ref[...]) + b_ref[...],
        0.0)
    y_ref[...] = y

    @pl.when(pl.program_id(0) == 0)
    def _():
        st_ref[...] = jnp.zeros_like(st_ref)
    st_ref[0:1, :] += jnp.sum(y, axis=0, keepdims=True)
    st_ref[1:2, :] += jnp.sum(y * y, axis=0, keepdims=True)


_tc_conv_out = pl.pallas_call(
    _d1_body,
    grid=(NB,),
    in_specs=[
        pl.BlockSpec((BM, D), lambda i: (i, 0)),
        pl.BlockSpec((BM, D), lambda i: (i + NP // BM, 0)),
        pl.BlockSpec((BM, D), lambda i: (i, 0)),
        pl.BlockSpec((BM, 1), lambda i: (i, 0)),
        pl.BlockSpec((1, D), lambda i: (0, 0)),
    ],
    out_specs=[
        pl.BlockSpec((BM, D), lambda i: (i, 0)),
        pl.BlockSpec((8, D), lambda i: (0, 0)),
    ],
    out_shape=[
        jax.ShapeDtypeStruct((N, D), jnp.float32),
        jax.ShapeDtypeStruct((8, D), jnp.float32),
    ],
)


def _make_bn_apply(residual, matmul):
    def body(*refs):
        it = iter(refs)
        y_ref = next(it)
        st_ref = next(it)
        g_ref = next(it)
        be_ref = next(it)
        r_ref = next(it) if residual else None
        if matmul:
            dinv_ref = next(it)
            w_ref = next(it)
        x_ref = next(it)
        u_ref = next(it) if matmul else None

        mean = st_ref[0:1, :] * (1.0 / N)
        var = st_ref[1:2, :] * (1.0 / N) - mean * mean
        rstd = lax.rsqrt(var + 1e-5)
        xl = (y_ref[...] - mean) * rstd * g_ref[...] + be_ref[...]
        if residual:
            xl = xl + r_ref[...]
        x_ref[...] = xl
        if matmul:
            h = jnp.dot(xl, w_ref[...], preferred_element_type=jnp.float32)
            u_ref[...] = h * dinv_ref[...]

    in_specs = [
        pl.BlockSpec((BM, D), lambda i: (i, 0)),
        pl.BlockSpec((8, D), lambda i: (0, 0)),
        pl.BlockSpec((1, D), lambda i: (0, 0)),
        pl.BlockSpec((1, D), lambda i: (0, 0)),
    ]
    if residual:
        in_specs.append(pl.BlockSpec((BM, D), lambda i: (i, 0)))
    if matmul:
        in_specs.append(pl.BlockSpec((BM, 1), lambda i: (i, 0)))
        in_specs.append(pl.BlockSpec((D, D), lambda i: (0, 0)))
    out_specs = [pl.BlockSpec((BM, D), lambda i: (i, 0))]
    out_shape = [jax.ShapeDtypeStruct((N, D), jnp.float32)]
    if matmul:
        out_specs.append(pl.BlockSpec((BM, D), lambda i: (i, 0)))
        out_shape.append(jax.ShapeDtypeStruct((N, D), jnp.float32))
    return pl.pallas_call(body, grid=(NB,), in_specs=in_specs,
                          out_specs=out_specs, out_shape=out_shape)


_tc_bn_mm = _make_bn_apply(residual=False, matmul=True)
_tc_bn_res_mm = _make_bn_apply(residual=True, matmul=True)
_tc_bn_res = _make_bn_apply(residual=True, matmul=False)


def _head_body(s0_ref, s1_ref, c0_ref, c1_ref, lw1_ref, lb1_ref,
               lw2_ref, lb2_ref, out_ref):
    cnt = c0_ref[:, 0:1] + c1_ref[:, 0:1]
    pooled = (s0_ref[...] + s1_ref[...]) / jnp.maximum(cnt, 1.0)
    h = jnp.maximum(
        jnp.dot(pooled, lw1_ref[...], preferred_element_type=jnp.float32)
        + lb1_ref[...], 0.0)
    out_ref[...] = (
        jnp.dot(h, lw2_ref[...], preferred_element_type=jnp.float32)
        + lb2_ref[...])


_tc_head = pl.pallas_call(
    _head_body,
    grid=(1,),
    in_specs=[
        pl.BlockSpec((G, D), lambda i: (0, 0)),
        pl.BlockSpec((G, D), lambda i: (GP // G, 0)),
        pl.BlockSpec((G, D), lambda i: (0, 0)),
        pl.BlockSpec((G, D), lambda i: (GP // G, 0)),
        pl.BlockSpec((D, D), lambda i: (0, 0)),
        pl.BlockSpec((1, D), lambda i: (0, 0)),
        pl.BlockSpec((D, 1), lambda i: (0, 0)),
        pl.BlockSpec((1, 1), lambda i: (0, 0)),
    ],
    out_specs=pl.BlockSpec((G, 1), lambda i: (0, 0)),
    out_shape=jax.ShapeDtypeStruct((G, 1), jnp.float32),
)


def kernel(x, edge_index, batch, W1, b1, W2, b2, W3, b3,
           g1, be1, g2, be2, g3, be3, lw1, lb1, lw2, lb2):
    src = edge_index[0]
    dst = edge_index[1]
    pad_e = EPAD - E
    src_pad = jnp.concatenate([src, jnp.zeros((pad_e,), jnp.int32)])
    dst_pad = jnp.concatenate([dst, jnp.full((pad_e,), N, jnp.int32)])
    batch_pad = jnp.concatenate([batch, jnp.full((NBP - N,), G, jnp.int32)])
    rowids = jnp.concatenate([jnp.arange(N, dtype=jnp.int32),
                              jnp.zeros((NBP - N,), jnp.int32)])
    zeros128 = jnp.zeros((NP, D), jnp.float32)
    ones128 = jnp.ones((CH, D), jnp.float32)

    b1r, b2r, b3r = (v.reshape(1, D) for v in (b1, b2, b3))
    g1r, g2r, g3r = (v.reshape(1, D) for v in (g1, g2, g3))
    be1r, be2r, be3r = (v.reshape(1, D) for v in (be1, be2, be3))
    lb1r = lb1.reshape(1, D)
    lb2r = lb2.reshape(1, 1)

    degp, cntp = _sc_degree(dst_pad, batch_pad, ones128, zeros128)
    dinv, u1 = _tc_prep(x, W1, degp, degp)

    p1 = _sc_aggregate(u1, src_pad, dst_pad, zeros128)
    y1, st1 = _tc_conv_out(p1, p1, u1, dinv, b1r)
    x1, u2 = _tc_bn_mm(y1, st1, g1r, be1r, dinv, W2)

    p2 = _sc_aggregate(u2, src_pad, dst_pad, zeros128)
    y2, st2 = _tc_conv_out(p2, p2, u2, dinv, b2r)
    x2, u3 = _tc_bn_res_mm(y2, st2, g2r, be2r, x1, dinv, W3)

    p3 = _sc_aggregate(u3, src_pad, dst_pad, zeros128)
    y3, st3 = _tc_conv_out(p3, p3, u3, dinv, b3r)
    (x3,) = _tc_bn_res(y3, st3, g3r, be3r, x2)

    poolp = _sc_pool(x3, rowids, batch_pad, zeros128)
    return _tc_head(poolp, poolp, cntp, cntp, lw1, lb1r, lw2, lb2r)
